# baseline (speedup 1.0000x reference)
import jax
import jax.numpy as jnp
from jax.experimental import pallas as pl
from jax.experimental.pallas import tpu as pltpu

N_POS = 64
BOOK = 8192
DIM = 32
BATCH = 16


def _argmin_body(z2_ref, zt_ref, cb_ref, word_ref, wq_ref, idx_ref):
    zb = zt_ref[0]
    cb = cb_ref[0]
    z2 = z2_ref[0, 0, :].reshape(BATCH, 1)
    zc = jax.lax.dot_general(zb, cb, (((1,), (1,)), ((), ())))
    c2 = jnp.sum(cb * cb, axis=1).reshape(1, BOOK)
    dist = (z2 + c2) - 2.0 * zc
    m = jnp.min(dist, axis=1, keepdims=True)
    iota = jax.lax.broadcasted_iota(jnp.int32, (BATCH, BOOK), 1)
    idx = jnp.min(jnp.where(dist == m, iota, jnp.int32(BOOK)), axis=1)
    one_hot = (iota == idx[:, None]).astype(jnp.float32)
    wq = jax.lax.dot_general(one_hot, cb, (((1,), (0,)), ((), ())),
                             precision=jax.lax.Precision.HIGHEST)
    wq_ref[0] = wq
    word_ref[0] = zb + (wq - zb)
    idx_ref[0, 0, :] = idx


def _onehot_body(idx_ref, oh_ref):
    g = pl.program_id(0)
    idx = idx_ref[...]
    iota = jax.lax.broadcasted_iota(jnp.int32, (BATCH, N_POS, 1024), 2)
    oh_ref[...] = (idx[:, :, None] == iota + g * 1024).astype(jnp.float32)


def kernel(z, codebook):
    zt = jnp.transpose(z, (1, 0, 2))
    z2 = jnp.sum(z * z, axis=-1)
    z2t = jnp.transpose(z2, (1, 0)).reshape(N_POS, 1, BATCH)

    word_t, wq_t, idx3 = pl.pallas_call(
        _argmin_body,
        grid=(N_POS,),
        in_specs=[
            pl.BlockSpec((1, 1, BATCH), lambda n: (n, 0, 0)),
            pl.BlockSpec((1, BATCH, DIM), lambda n: (n, 0, 0)),
            pl.BlockSpec((1, BOOK, DIM), lambda n: (n, 0, 0)),
        ],
        out_specs=[
            pl.BlockSpec((1, BATCH, DIM), lambda n: (n, 0, 0)),
            pl.BlockSpec((1, BATCH, DIM), lambda n: (n, 0, 0)),
            pl.BlockSpec((1, 1, BATCH), lambda n: (n, 0, 0)),
        ],
        out_shape=[
            jax.ShapeDtypeStruct((N_POS, BATCH, DIM), jnp.float32),
            jax.ShapeDtypeStruct((N_POS, BATCH, DIM), jnp.float32),
            jax.ShapeDtypeStruct((N_POS, 1, BATCH), jnp.int32),
        ],
        compiler_params=pltpu.CompilerParams(
            dimension_semantics=("arbitrary",),
        ),
    )(z2t, zt, codebook)

    idx = jnp.transpose(idx3.reshape(N_POS, BATCH), (1, 0))

    one_hot = pl.pallas_call(
        _onehot_body,
        grid=(BOOK // 1024,),
        in_specs=[pl.BlockSpec((BATCH, N_POS), lambda g: (0, 0))],
        out_specs=pl.BlockSpec((BATCH, N_POS, 1024), lambda g: (0, 0, g)),
        out_shape=jax.ShapeDtypeStruct((BATCH, N_POS, BOOK), jnp.float32),
        compiler_params=pltpu.CompilerParams(
            dimension_semantics=("arbitrary",),
        ),
    )(idx)

    word = jnp.transpose(word_t, (1, 0, 2))
    wq = jnp.transpose(wq_t, (1, 0, 2))
    return (word, wq, idx, one_hot)


# speedup vs baseline: 4.3620x; 4.3620x over previous
import jax
import jax.numpy as jnp
from jax.experimental import pallas as pl
from jax.experimental.pallas import tpu as pltpu

N_POS = 64
BOOK = 8192
DIM = 32
BATCH = 16


def _argmin_body(z2_ref, zt_ref, cbt_ref, word_ref, wq_ref, idx_ref):
    zb = zt_ref[0]
    cbt = cbt_ref[0]
    z2 = z2_ref[0, 0, :].reshape(BATCH, 1)
    zc = jax.lax.dot_general(zb, cbt, (((1,), (0,)), ((), ())))
    c2 = jnp.sum(cbt * cbt, axis=0).reshape(1, BOOK)
    dist = (z2 + c2) - 2.0 * zc
    m = jnp.min(dist, axis=1, keepdims=True)
    iota = jax.lax.broadcasted_iota(jnp.int32, (BATCH, BOOK), 1)
    idx = jnp.min(jnp.where(dist == m, iota, jnp.int32(BOOK)), axis=1)
    one_hot = (iota == idx[:, None]).astype(jnp.float32)
    wq = jax.lax.dot_general(one_hot, cbt, (((1,), (1,)), ((), ())))
    wq_ref[0] = wq
    word_ref[0] = zb + (wq - zb)
    idx_ref[0, 0, :] = idx


def _onehot_body(idx_ref, oh_ref):
    g = pl.program_id(0)
    idx = idx_ref[...]
    iota = jax.lax.broadcasted_iota(jnp.int32, (BATCH, N_POS, 1024), 2)
    oh_ref[...] = (idx[:, :, None] == iota + g * 1024).astype(jnp.float32)


def kernel(z, codebook):
    cbt = jnp.transpose(codebook, (0, 2, 1))
    zt = jnp.transpose(z, (1, 0, 2))
    z2 = jnp.sum(z * z, axis=-1)
    z2t = jnp.transpose(z2, (1, 0)).reshape(N_POS, 1, BATCH)

    word_t, wq_t, idx3 = pl.pallas_call(
        _argmin_body,
        grid=(N_POS,),
        in_specs=[
            pl.BlockSpec((1, 1, BATCH), lambda n: (n, 0, 0)),
            pl.BlockSpec((1, BATCH, DIM), lambda n: (n, 0, 0)),
            pl.BlockSpec((1, DIM, BOOK), lambda n: (n, 0, 0)),
        ],
        out_specs=[
            pl.BlockSpec((1, BATCH, DIM), lambda n: (n, 0, 0)),
            pl.BlockSpec((1, BATCH, DIM), lambda n: (n, 0, 0)),
            pl.BlockSpec((1, 1, BATCH), lambda n: (n, 0, 0)),
        ],
        out_shape=[
            jax.ShapeDtypeStruct((N_POS, BATCH, DIM), jnp.float32),
            jax.ShapeDtypeStruct((N_POS, BATCH, DIM), jnp.float32),
            jax.ShapeDtypeStruct((N_POS, 1, BATCH), jnp.int32),
        ],
        compiler_params=pltpu.CompilerParams(
            dimension_semantics=("arbitrary",),
        ),
    )(z2t, zt, cbt)

    idx = jnp.transpose(idx3.reshape(N_POS, BATCH), (1, 0))

    one_hot = pl.pallas_call(
        _onehot_body,
        grid=(BOOK // 1024,),
        in_specs=[pl.BlockSpec((BATCH, N_POS), lambda g: (0, 0))],
        out_specs=pl.BlockSpec((BATCH, N_POS, 1024), lambda g: (0, 0, g)),
        out_shape=jax.ShapeDtypeStruct((BATCH, N_POS, BOOK), jnp.float32),
        compiler_params=pltpu.CompilerParams(
            dimension_semantics=("arbitrary",),
        ),
    )(idx)

    word = jnp.transpose(word_t, (1, 0, 2))
    wq = jnp.transpose(wq_t, (1, 0, 2))
    return (word, wq, idx, one_hot)


# speedup vs baseline: 7.2435x; 1.6606x over previous
import jax
import jax.numpy as jnp
from jax.experimental import pallas as pl
from jax.experimental.pallas import tpu as pltpu

N_POS = 64
BOOK = 8192
DIM = 32
BATCH = 16
P = 8


def _body(z2_ref, zt_ref, cbt_ref, word_ref, wq_ref, idx_ref, oh_ref):
    for p in range(P):
        zb = zt_ref[p]
        cbt = cbt_ref[p]
        z2 = z2_ref[p, 0, :].reshape(BATCH, 1)
        zc = jax.lax.dot_general(zb, cbt, (((1,), (0,)), ((), ())))
        c2 = jnp.sum(cbt * cbt, axis=0).reshape(1, BOOK)
        dist = (z2 + c2) - 2.0 * zc
        m = jnp.min(dist, axis=1, keepdims=True)
        iota = jax.lax.broadcasted_iota(jnp.int32, (BATCH, BOOK), 1)
        idx = jnp.min(jnp.where(dist == m, iota, jnp.int32(BOOK)), axis=1)
        one_hot = (iota == idx[:, None]).astype(jnp.float32)
        wq = jax.lax.dot_general(one_hot, cbt, (((1,), (1,)), ((), ())))
        wq_ref[p] = wq
        word_ref[p] = zb + (wq - zb)
        idx_ref[p, 0, :] = idx
        oh_ref[:, p, :] = one_hot


def kernel(z, codebook):
    cbt = jnp.transpose(codebook, (0, 2, 1))
    zt = jnp.transpose(z, (1, 0, 2))
    z2 = jnp.sum(z * z, axis=-1)
    z2t = jnp.transpose(z2, (1, 0)).reshape(N_POS, 1, BATCH)

    word_t, wq_t, idx3, one_hot = pl.pallas_call(
        _body,
        grid=(N_POS // P,),
        in_specs=[
            pl.BlockSpec((P, 1, BATCH), lambda g: (g, 0, 0)),
            pl.BlockSpec((P, BATCH, DIM), lambda g: (g, 0, 0)),
            pl.BlockSpec((P, DIM, BOOK), lambda g: (g, 0, 0)),
        ],
        out_specs=[
            pl.BlockSpec((P, BATCH, DIM), lambda g: (g, 0, 0)),
            pl.BlockSpec((P, BATCH, DIM), lambda g: (g, 0, 0)),
            pl.BlockSpec((P, 1, BATCH), lambda g: (g, 0, 0)),
            pl.BlockSpec((BATCH, P, BOOK), lambda g: (0, g, 0)),
        ],
        out_shape=[
            jax.ShapeDtypeStruct((N_POS, BATCH, DIM), jnp.float32),
            jax.ShapeDtypeStruct((N_POS, BATCH, DIM), jnp.float32),
            jax.ShapeDtypeStruct((N_POS, 1, BATCH), jnp.int32),
            jax.ShapeDtypeStruct((BATCH, N_POS, BOOK), jnp.float32),
        ],
        compiler_params=pltpu.CompilerParams(
            dimension_semantics=("arbitrary",),
        ),
    )(z2t, zt, cbt)

    idx = jnp.transpose(idx3.reshape(N_POS, BATCH), (1, 0))
    word = jnp.transpose(word_t, (1, 0, 2))
    wq = jnp.transpose(wq_t, (1, 0, 2))
    return (word, wq, idx, one_hot)


# speedup vs baseline: 7.4013x; 1.0218x over previous
import jax
import jax.numpy as jnp
from jax.experimental import pallas as pl
from jax.experimental.pallas import tpu as pltpu

N_POS = 64
BOOK = 8192
DIM = 32
BATCH = 16
P = 8


def _body(z2_ref, zt_ref, cbt_ref, word_ref, wq_ref, idx_ref, oh_ref):
    g = pl.program_id(0)
    for p in range(P):
        n = g * P + p
        zb = zt_ref[n]
        cbt = cbt_ref[p]
        z2 = z2_ref[n, 0, :].reshape(BATCH, 1)
        zc = jax.lax.dot_general(zb, cbt, (((1,), (0,)), ((), ())))
        c2 = jnp.sum(cbt * cbt, axis=0).reshape(1, BOOK)
        dist = (z2 + c2) - 2.0 * zc
        m = jnp.min(dist, axis=1, keepdims=True)
        iota = jax.lax.broadcasted_iota(jnp.int32, (BATCH, BOOK), 1)
        idx = jnp.min(jnp.where(dist == m, iota, jnp.int32(BOOK)), axis=1)
        one_hot = (iota == idx[:, None]).astype(jnp.float32)
        wq = jax.lax.dot_general(one_hot, cbt, (((1,), (1,)), ((), ())))
        wq_ref[n] = wq
        word_ref[n] = zb + (wq - zb)
        idx_ref[n, 0, :] = idx
        oh_ref[:, p, :] = one_hot


def kernel(z, codebook):
    cbt = jnp.transpose(codebook, (0, 2, 1))
    zt = jnp.transpose(z, (1, 0, 2))
    z2 = jnp.sum(z * z, axis=-1)
    z2t = jnp.transpose(z2, (1, 0)).reshape(N_POS, 1, BATCH)

    word_t, wq_t, idx3, one_hot = pl.pallas_call(
        _body,
        grid=(N_POS // P,),
        in_specs=[
            pl.BlockSpec((N_POS, 1, BATCH), lambda g: (0, 0, 0)),
            pl.BlockSpec((N_POS, BATCH, DIM), lambda g: (0, 0, 0)),
            pl.BlockSpec((P, DIM, BOOK), lambda g: (g, 0, 0)),
        ],
        out_specs=[
            pl.BlockSpec((N_POS, BATCH, DIM), lambda g: (0, 0, 0)),
            pl.BlockSpec((N_POS, BATCH, DIM), lambda g: (0, 0, 0)),
            pl.BlockSpec((N_POS, 1, BATCH), lambda g: (0, 0, 0)),
            pl.BlockSpec((BATCH, P, BOOK), lambda g: (0, g, 0)),
        ],
        out_shape=[
            jax.ShapeDtypeStruct((N_POS, BATCH, DIM), jnp.float32),
            jax.ShapeDtypeStruct((N_POS, BATCH, DIM), jnp.float32),
            jax.ShapeDtypeStruct((N_POS, 1, BATCH), jnp.int32),
            jax.ShapeDtypeStruct((BATCH, N_POS, BOOK), jnp.float32),
        ],
        compiler_params=pltpu.CompilerParams(
            dimension_semantics=("arbitrary",),
        ),
    )(z2t, zt, cbt)

    idx = jnp.transpose(idx3.reshape(N_POS, BATCH), (1, 0))
    word = jnp.transpose(word_t, (1, 0, 2))
    wq = jnp.transpose(wq_t, (1, 0, 2))
    return (word, wq, idx, one_hot)
